# Initial kernel scaffold; baseline (speedup 1.0000x reference)
import jax
import jax.numpy as jnp
from jax.experimental import pallas as pl


def kernel(x, edge_index, W1, b1, W2, b2):
    raise NotImplementedError("write your pallas kernel here")


# speedup vs baseline: 10.5899x; 10.5899x over previous
import jax
import jax.numpy as jnp
from jax import lax
from jax.experimental import pallas as pl
from jax.experimental.pallas import tpu as pltpu
from jax.experimental.pallas import tpu_sc as plsc

N_NODES = 10000
N_EDGES = 320000
D_IN = 128
D_HID = 256
D_OUT = 128

N_TILES = 16
N_SC = 2
NW = N_TILES * N_SC
CHUNK = 128
ROWS_PER_TILE = 632
N_PAD = N_TILES * ROWS_PER_TILE
CHUNKS_PER_TILE = -(-N_EDGES // (NW * CHUNK))
E_PAD = NW * CHUNK * CHUNKS_PER_TILE

_MESH = plsc.VectorSubcoreMesh(core_axis_name="c", subcore_axis_name="s")

_SEGS = [(0, 128), (128, 128), (256, 128), (384, 128), (512, 120)]


def _bounce(src_ref, dst_ref, bounce_ref, rbase):
    for s_off, s_len in _SEGS:
        sl = pl.ds(rbase + s_off, s_len)
        pltpu.sync_copy(src_ref.at[sl], bounce_ref.at[pl.ds(0, s_len)])
        pltpu.sync_copy(bounce_ref.at[pl.ds(0, s_len)], dst_ref.at[sl])


def _agg_body(feat, src, dst, zf,
              agg0_o, agg1_o,
              src_v, dst_v, rows_v, agg_sh):
    cid = lax.axis_index("c")
    tid = lax.axis_index("s")
    wid = tid * N_SC + cid
    rbase = tid * ROWS_PER_TILE
    _bounce(zf, agg_sh, rows_v, rbase)
    plsc.subcore_barrier()

    ebase = wid * (CHUNKS_PER_TILE * CHUNK)

    def step(k, carry):
        off = ebase + k * CHUNK
        pltpu.sync_copy(src.at[pl.ds(off, CHUNK)], src_v)
        pltpu.sync_copy(dst.at[pl.ds(off, CHUNK)], dst_v)
        pltpu.sync_copy(feat.at[src_v], rows_v)
        pltpu.sync_copy(rows_v, agg_sh.at[dst_v], add=True)
        return carry

    lax.fori_loop(0, CHUNKS_PER_TILE, step, None)
    plsc.subcore_barrier()

    @pl.when(cid == 0)
    def _():
        _bounce(agg_sh, agg0_o, rows_v, rbase)

    @pl.when(cid == 1)
    def _():
        _bounce(agg_sh, agg1_o, rows_v, rbase)


def _deg_body(dst, zd,
              deg0_o, deg1_o,
              dst_v, ones_v, deg_sh):
    cid = lax.axis_index("c")
    tid = lax.axis_index("s")
    wid = tid * N_SC + cid
    rbase = tid * ROWS_PER_TILE
    _bounce(zd, deg_sh, ones_v, rbase)
    for j in range(CHUNK // 16):
        ones_v[pl.ds(j * 16, 16)] = jnp.ones((16,), jnp.float32)
    plsc.subcore_barrier()

    ebase = wid * (CHUNKS_PER_TILE * CHUNK)

    def step(k, carry):
        off = ebase + k * CHUNK
        pltpu.sync_copy(dst.at[pl.ds(off, CHUNK)], dst_v)
        pltpu.sync_copy(ones_v, deg_sh.at[dst_v], add=True)
        return carry

    lax.fori_loop(0, CHUNKS_PER_TILE, step, None)
    plsc.subcore_barrier()

    @pl.when(cid == 0)
    def _():
        _bounce(deg_sh, deg0_o, ones_v, rbase)

    @pl.when(cid == 1)
    def _():
        _bounce(deg_sh, deg1_o, ones_v, rbase)


_agg_call = pl.kernel(
    _agg_body,
    out_type=[
        jax.ShapeDtypeStruct((N_PAD, D_IN), jnp.float32),
        jax.ShapeDtypeStruct((N_PAD, D_IN), jnp.float32),
    ],
    mesh=_MESH,
    scratch_types=[
        pltpu.VMEM((CHUNK,), jnp.int32),
        pltpu.VMEM((CHUNK,), jnp.int32),
        pltpu.VMEM((CHUNK, D_IN), jnp.float32),
        pltpu.VMEM_SHARED((N_PAD, D_IN), jnp.float32),
    ],
    name="sage_agg",
)

_deg_call = pl.kernel(
    _deg_body,
    out_type=[
        jax.ShapeDtypeStruct((N_PAD,), jnp.float32),
        jax.ShapeDtypeStruct((N_PAD,), jnp.float32),
    ],
    mesh=_MESH,
    scratch_types=[
        pltpu.VMEM((CHUNK,), jnp.int32),
        pltpu.VMEM((CHUNK,), jnp.float32),
        pltpu.VMEM_SHARED((N_PAD,), jnp.float32),
    ],
    name="sage_deg",
)


_BM = 1264
_GRID = N_PAD // _BM


def _mlp_body(a0, a1, xr, d0, d1, w1, b1r, w2, g_o):
    recip = 1.0 / (d0[...] + d1[...] + 1.0)
    mean1 = (a0[...] + a1[...] + xr[...]) * recip
    h = jnp.dot(mean1, w1[...], preferred_element_type=jnp.float32) + b1r[...]
    h = jnp.maximum(h, 0.0)
    g_o[...] = jnp.dot(h, w2[...], preferred_element_type=jnp.float32)


def _mlp(agg0, agg1, x_p, d0, d1, W1, b1r, W2):
    return pl.pallas_call(
        _mlp_body,
        grid=(_GRID,),
        in_specs=[
            pl.BlockSpec((_BM, D_IN), lambda i: (i, 0)),
            pl.BlockSpec((_BM, D_IN), lambda i: (i, 0)),
            pl.BlockSpec((_BM, D_IN), lambda i: (i, 0)),
            pl.BlockSpec((_BM, 1), lambda i: (i, 0)),
            pl.BlockSpec((_BM, 1), lambda i: (i, 0)),
            pl.BlockSpec((D_IN, D_HID), lambda i: (0, 0)),
            pl.BlockSpec((1, D_HID), lambda i: (0, 0)),
            pl.BlockSpec((D_HID, D_OUT), lambda i: (0, 0)),
        ],
        out_specs=pl.BlockSpec((_BM, D_OUT), lambda i: (i, 0)),
        out_shape=jax.ShapeDtypeStruct((N_PAD, D_OUT), jnp.float32),
    )(agg0, agg1, x_p, d0, d1, W1, b1r, W2)


def _final_body(a0, a1, gr, d0, d1, b2r, o):
    recip = 1.0 / (d0[...] + d1[...] + 1.0)
    z = (a0[...] + a1[...] + gr[...]) * recip + b2r[...]
    m = jnp.max(z, axis=1, keepdims=True)
    zs = z - m
    lse = jnp.log(jnp.sum(jnp.exp(zs), axis=1, keepdims=True))
    o[...] = zs - lse


def _final(a0, a1, g, d0, d1, b2r):
    return pl.pallas_call(
        _final_body,
        grid=(_GRID,),
        in_specs=[
            pl.BlockSpec((_BM, D_OUT), lambda i: (i, 0)),
            pl.BlockSpec((_BM, D_OUT), lambda i: (i, 0)),
            pl.BlockSpec((_BM, D_OUT), lambda i: (i, 0)),
            pl.BlockSpec((_BM, 1), lambda i: (i, 0)),
            pl.BlockSpec((_BM, 1), lambda i: (i, 0)),
            pl.BlockSpec((1, D_OUT), lambda i: (0, 0)),
        ],
        out_specs=pl.BlockSpec((_BM, D_OUT), lambda i: (i, 0)),
        out_shape=jax.ShapeDtypeStruct((N_PAD, D_OUT), jnp.float32),
    )(a0, a1, g, d0, d1, b2r)


def kernel(x, edge_index, W1, b1, W2, b2):
    src = edge_index[0]
    dst = edge_index[1]
    pad = E_PAD - N_EDGES
    fill = jnp.arange(pad, dtype=jnp.int32) % 16
    src_p = jnp.concatenate([src, fill])
    dst_p = jnp.concatenate([dst, N_NODES + fill])
    zf = jnp.zeros((N_PAD, D_IN), jnp.float32)
    zd = jnp.zeros((N_PAD,), jnp.float32)

    agg0, agg1 = _agg_call(x, src_p, dst_p, zf)
    deg0, deg1 = _deg_call(dst_p, zd)
    d0 = deg0[:, None]
    d1 = deg1[:, None]
    x_p = jnp.pad(x, ((0, N_PAD - N_NODES), (0, 0)))

    g = _mlp(agg0, agg1, x_p, d0, d1, W1, b1.reshape(1, -1), W2)

    agg2_0, agg2_1 = _agg_call(g, src_p, dst_p, zf)

    out = _final(agg2_0, agg2_1, g, d0, d1, b2.reshape(1, -1))
    return out[:N_NODES]


# speedup vs baseline: 15.5854x; 1.4717x over previous
import jax
import jax.numpy as jnp
from jax import lax
from jax.experimental import pallas as pl
from jax.experimental.pallas import tpu as pltpu
from jax.experimental.pallas import tpu_sc as plsc

N_NODES = 10000
N_EDGES = 320000
D_IN = 128
D_HID = 256
D_OUT = 128

N_TILES = 16
N_SC = 2
NW = N_TILES * N_SC
CHUNK = 128
ROWS_PER_TILE = 632
N_PAD = N_TILES * ROWS_PER_TILE
NBUF = 2
GROUPS = 40
CHUNKS_PER_TILE = NBUF * GROUPS
E_PAD = NW * CHUNK * CHUNKS_PER_TILE

_MESH = plsc.VectorSubcoreMesh(core_axis_name="c", subcore_axis_name="s")

_SEGS = [(0, 128), (128, 128), (256, 128), (384, 128), (512, 120)]


def _bounce(src_ref, dst_ref, bounce_ref, rbase):
    for s_off, s_len in _SEGS:
        sl = pl.ds(rbase + s_off, s_len)
        pltpu.sync_copy(src_ref.at[sl], bounce_ref.at[pl.ds(0, s_len)])
        pltpu.sync_copy(bounce_ref.at[pl.ds(0, s_len)], dst_ref.at[sl])


def _vcopy_row(src2d, row, dst1d):
    for j in range(CHUNK // 16):
        dst1d[pl.ds(j * 16, 16)] = src2d[row, pl.ds(j * 16, 16)]


def _agg_body(feat, e3, zf,
              agg0_o, agg1_o,
              idx, gidx, sidx, rows, si, sg, ss, agg_sh):
    cid = lax.axis_index("c")
    tid = lax.axis_index("s")
    wid = tid * N_SC + cid
    rbase = tid * ROWS_PER_TILE
    _bounce(zf, agg_sh, rows[0], rbase)
    plsc.subcore_barrier()

    cbase = wid * CHUNKS_PER_TILE

    def _wait_idx(b):
        pltpu.make_async_copy(e3.at[0], idx[b], si[b]).wait()

    def _wait_gather(b):
        pltpu.make_async_copy(zf.at[pl.ds(0, CHUNK)], rows[b], sg[b]).wait()

    def _wait_scatter(b):
        pltpu.make_async_copy(zf.at[pl.ds(0, CHUNK)], rows[b], ss[b]).wait()

    def _group(g, first, issue_next):
        for b in range(NBUF):
            k = cbase + g * NBUF + b
            if not first:
                _wait_scatter(b)
            _wait_idx(b)
            _vcopy_row(idx[b], 0, gidx[b])
            _vcopy_row(idx[b], 1, sidx[b])
            if issue_next:
                pltpu.async_copy(e3.at[k + NBUF], idx[b], si[b])
            pltpu.async_copy(feat.at[gidx[b]], rows[b], sg[b])
        for b in range(NBUF):
            _wait_gather(b)
            pltpu.async_copy(rows[b], agg_sh.at[sidx[b]], ss[b], add=True)

    for b in range(NBUF):
        pltpu.async_copy(e3.at[cbase + b], idx[b], si[b])
    _group(0, True, True)
    lax.fori_loop(1, GROUPS - 1,
                  lambda g, c: (_group(g, False, True), c)[1], None)
    _group(GROUPS - 1, False, False)
    for b in range(NBUF):
        _wait_scatter(b)
    plsc.subcore_barrier()

    @pl.when(cid == 0)
    def _():
        _bounce(agg_sh, agg0_o, rows[0], rbase)

    @pl.when(cid == 1)
    def _():
        _bounce(agg_sh, agg1_o, rows[0], rbase)


def _deg_body(dst, zd,
              deg0_o, deg1_o,
              dst_v, ones_v, deg_sh):
    cid = lax.axis_index("c")
    tid = lax.axis_index("s")
    wid = tid * N_SC + cid
    rbase = tid * ROWS_PER_TILE
    _bounce(zd, deg_sh, ones_v, rbase)
    for j in range(CHUNK // 16):
        ones_v[pl.ds(j * 16, 16)] = jnp.ones((16,), jnp.float32)
    plsc.subcore_barrier()

    ebase = wid * (CHUNKS_PER_TILE * CHUNK)

    def step(k, carry):
        off = ebase + k * CHUNK
        pltpu.sync_copy(dst.at[pl.ds(off, CHUNK)], dst_v)
        pltpu.sync_copy(ones_v, deg_sh.at[dst_v], add=True)
        return carry

    lax.fori_loop(0, CHUNKS_PER_TILE, step, None)
    plsc.subcore_barrier()

    @pl.when(cid == 0)
    def _():
        _bounce(deg_sh, deg0_o, ones_v, rbase)

    @pl.when(cid == 1)
    def _():
        _bounce(deg_sh, deg1_o, ones_v, rbase)


_agg_call = pl.kernel(
    _agg_body,
    out_type=[
        jax.ShapeDtypeStruct((N_PAD, D_IN), jnp.float32),
        jax.ShapeDtypeStruct((N_PAD, D_IN), jnp.float32),
    ],
    mesh=_MESH,
    scratch_types=[
        [pltpu.VMEM((2, CHUNK), jnp.int32) for _ in range(NBUF)],
        [pltpu.VMEM((CHUNK,), jnp.int32) for _ in range(NBUF)],
        [pltpu.VMEM((CHUNK,), jnp.int32) for _ in range(NBUF)],
        [pltpu.VMEM((CHUNK, D_IN), jnp.float32) for _ in range(NBUF)],
        [pltpu.SemaphoreType.DMA for _ in range(NBUF)],
        [pltpu.SemaphoreType.DMA for _ in range(NBUF)],
        [pltpu.SemaphoreType.DMA for _ in range(NBUF)],
        pltpu.VMEM_SHARED((N_PAD, D_IN), jnp.float32),
    ],
    name="sage_agg",
)

_deg_call = pl.kernel(
    _deg_body,
    out_type=[
        jax.ShapeDtypeStruct((N_PAD,), jnp.float32),
        jax.ShapeDtypeStruct((N_PAD,), jnp.float32),
    ],
    mesh=_MESH,
    scratch_types=[
        pltpu.VMEM((CHUNK,), jnp.int32),
        pltpu.VMEM((CHUNK,), jnp.float32),
        pltpu.VMEM_SHARED((N_PAD,), jnp.float32),
    ],
    name="sage_deg",
)


_BM = 1264
_GRID = N_PAD // _BM


def _mlp_body(a0, a1, xr, d0, d1, w1, b1r, w2, g_o):
    recip = 1.0 / (d0[...] + d1[...] + 1.0)
    mean1 = (a0[...] + a1[...] + xr[...]) * recip
    h = jnp.dot(mean1, w1[...], preferred_element_type=jnp.float32) + b1r[...]
    h = jnp.maximum(h, 0.0)
    g_o[...] = jnp.dot(h, w2[...], preferred_element_type=jnp.float32)


def _mlp(agg0, agg1, x_p, d0, d1, W1, b1r, W2):
    return pl.pallas_call(
        _mlp_body,
        grid=(_GRID,),
        in_specs=[
            pl.BlockSpec((_BM, D_IN), lambda i: (i, 0)),
            pl.BlockSpec((_BM, D_IN), lambda i: (i, 0)),
            pl.BlockSpec((_BM, D_IN), lambda i: (i, 0)),
            pl.BlockSpec((_BM, 1), lambda i: (i, 0)),
            pl.BlockSpec((_BM, 1), lambda i: (i, 0)),
            pl.BlockSpec((D_IN, D_HID), lambda i: (0, 0)),
            pl.BlockSpec((1, D_HID), lambda i: (0, 0)),
            pl.BlockSpec((D_HID, D_OUT), lambda i: (0, 0)),
        ],
        out_specs=pl.BlockSpec((_BM, D_OUT), lambda i: (i, 0)),
        out_shape=jax.ShapeDtypeStruct((N_PAD, D_OUT), jnp.float32),
    )(agg0, agg1, x_p, d0, d1, W1, b1r, W2)


def _final_body(a0, a1, gr, d0, d1, b2r, o):
    recip = 1.0 / (d0[...] + d1[...] + 1.0)
    z = (a0[...] + a1[...] + gr[...]) * recip + b2r[...]
    m = jnp.max(z, axis=1, keepdims=True)
    zs = z - m
    lse = jnp.log(jnp.sum(jnp.exp(zs), axis=1, keepdims=True))
    o[...] = zs - lse


def _final(a0, a1, g, d0, d1, b2r):
    return pl.pallas_call(
        _final_body,
        grid=(_GRID,),
        in_specs=[
            pl.BlockSpec((_BM, D_OUT), lambda i: (i, 0)),
            pl.BlockSpec((_BM, D_OUT), lambda i: (i, 0)),
            pl.BlockSpec((_BM, D_OUT), lambda i: (i, 0)),
            pl.BlockSpec((_BM, 1), lambda i: (i, 0)),
            pl.BlockSpec((_BM, 1), lambda i: (i, 0)),
            pl.BlockSpec((1, D_OUT), lambda i: (0, 0)),
        ],
        out_specs=pl.BlockSpec((_BM, D_OUT), lambda i: (i, 0)),
        out_shape=jax.ShapeDtypeStruct((N_PAD, D_OUT), jnp.float32),
    )(a0, a1, g, d0, d1, b2r)


def kernel(x, edge_index, W1, b1, W2, b2):
    src = edge_index[0]
    dst = edge_index[1]
    pad = E_PAD - N_EDGES
    fill = jnp.arange(pad, dtype=jnp.int32)
    src_p = jnp.concatenate([src, fill % N_NODES])
    dst_p = jnp.concatenate([dst, N_NODES + fill % (N_PAD - N_NODES)])
    e3 = jnp.stack([src_p.reshape(-1, CHUNK), dst_p.reshape(-1, CHUNK)],
                   axis=1)
    zf = jnp.zeros((N_PAD, D_IN), jnp.float32)
    zd = jnp.zeros((N_PAD,), jnp.float32)

    agg0, agg1 = _agg_call(x, e3, zf)
    deg0, deg1 = _deg_call(dst_p, zd)
    d0 = deg0[:, None]
    d1 = deg1[:, None]
    x_p = jnp.pad(x, ((0, N_PAD - N_NODES), (0, 0)))

    g = _mlp(agg0, agg1, x_p, d0, d1, W1, b1.reshape(1, -1), W2)

    agg2_0, agg2_1 = _agg_call(g, e3, zf)

    out = _final(agg2_0, agg2_1, g, d0, d1, b2.reshape(1, -1))
    return out[:N_NODES]


# speedup vs baseline: 16.1015x; 1.0331x over previous
import jax
import jax.numpy as jnp
from jax import lax
from jax.experimental import pallas as pl
from jax.experimental.pallas import tpu as pltpu
from jax.experimental.pallas import tpu_sc as plsc

N_NODES = 10000
N_EDGES = 320000
D_IN = 128
D_HID = 256
D_OUT = 128

N_TILES = 16
N_SC = 2
NW = N_TILES * N_SC
CHUNK = 64
ROWS_PER_TILE = 632
N_PAD = N_TILES * ROWS_PER_TILE
NBUF = 4
GROUPS = 40
CHUNKS_PER_TILE = NBUF * GROUPS
E_PAD = NW * CHUNK * CHUNKS_PER_TILE

_MESH = plsc.VectorSubcoreMesh(core_axis_name="c", subcore_axis_name="s")

_SEGS = [(o, min(CHUNK, ROWS_PER_TILE - o)) for o in range(0, ROWS_PER_TILE, CHUNK)]


def _bounce(src_ref, dst_ref, bounce_ref, rbase):
    for s_off, s_len in _SEGS:
        sl = pl.ds(rbase + s_off, s_len)
        pltpu.sync_copy(src_ref.at[sl], bounce_ref.at[pl.ds(0, s_len)])
        pltpu.sync_copy(bounce_ref.at[pl.ds(0, s_len)], dst_ref.at[sl])


def _vcopy_row(src2d, row, dst1d):
    for j in range(CHUNK // 16):
        dst1d[pl.ds(j * 16, 16)] = src2d[row, pl.ds(j * 16, 16)]


def _agg_body(feat, e3, zf,
              agg0_o, agg1_o,
              idx, gidx, sidx, rows, si, sg, ss, agg_sh):
    cid = lax.axis_index("c")
    tid = lax.axis_index("s")
    wid = tid * N_SC + cid
    rbase = tid * ROWS_PER_TILE
    _bounce(zf, agg_sh, rows[0], rbase)
    plsc.subcore_barrier()

    cbase = wid * CHUNKS_PER_TILE

    def _wait_idx(b):
        pltpu.make_async_copy(e3.at[0], idx[b], si[b]).wait()

    def _wait_gather(b):
        pltpu.make_async_copy(zf.at[pl.ds(0, CHUNK)], rows[b], sg[b]).wait()

    def _wait_scatter(b):
        pltpu.make_async_copy(zf.at[pl.ds(0, CHUNK)], rows[b], ss[b]).wait()

    def _group(g, first, issue_next):
        for b in range(NBUF):
            k = cbase + g * NBUF + b
            if not first:
                _wait_scatter(b)
            _wait_idx(b)
            _vcopy_row(idx[b], 0, gidx[b])
            _vcopy_row(idx[b], 1, sidx[b])
            if issue_next:
                pltpu.async_copy(e3.at[k + NBUF], idx[b], si[b])
            pltpu.async_copy(feat.at[gidx[b]], rows[b], sg[b])
        for b in range(NBUF):
            _wait_gather(b)
            pltpu.async_copy(rows[b], agg_sh.at[sidx[b]], ss[b], add=True)

    for b in range(NBUF):
        pltpu.async_copy(e3.at[cbase + b], idx[b], si[b])
    _group(0, True, True)
    lax.fori_loop(1, GROUPS - 1,
                  lambda g, c: (_group(g, False, True), c)[1], None)
    _group(GROUPS - 1, False, False)
    for b in range(NBUF):
        _wait_scatter(b)
    plsc.subcore_barrier()

    @pl.when(cid == 0)
    def _():
        _bounce(agg_sh, agg0_o, rows[0], rbase)

    @pl.when(cid == 1)
    def _():
        _bounce(agg_sh, agg1_o, rows[0], rbase)


def _deg_body(dst, zd,
              deg0_o, deg1_o,
              dst_v, ones_v, deg_sh):
    cid = lax.axis_index("c")
    tid = lax.axis_index("s")
    wid = tid * N_SC + cid
    rbase = tid * ROWS_PER_TILE
    _bounce(zd, deg_sh, ones_v, rbase)
    for j in range(CHUNK // 16):
        ones_v[pl.ds(j * 16, 16)] = jnp.ones((16,), jnp.float32)
    plsc.subcore_barrier()

    ebase = wid * (CHUNKS_PER_TILE * CHUNK)

    def step(k, carry):
        off = ebase + k * CHUNK
        pltpu.sync_copy(dst.at[pl.ds(off, CHUNK)], dst_v)
        pltpu.sync_copy(ones_v, deg_sh.at[dst_v], add=True)
        return carry

    lax.fori_loop(0, CHUNKS_PER_TILE, step, None)
    plsc.subcore_barrier()

    @pl.when(cid == 0)
    def _():
        _bounce(deg_sh, deg0_o, ones_v, rbase)

    @pl.when(cid == 1)
    def _():
        _bounce(deg_sh, deg1_o, ones_v, rbase)


_agg_call = pl.kernel(
    _agg_body,
    out_type=[
        jax.ShapeDtypeStruct((N_PAD, D_IN), jnp.float32),
        jax.ShapeDtypeStruct((N_PAD, D_IN), jnp.float32),
    ],
    mesh=_MESH,
    scratch_types=[
        [pltpu.VMEM((2, CHUNK), jnp.int32) for _ in range(NBUF)],
        [pltpu.VMEM((CHUNK,), jnp.int32) for _ in range(NBUF)],
        [pltpu.VMEM((CHUNK,), jnp.int32) for _ in range(NBUF)],
        [pltpu.VMEM((CHUNK, D_IN), jnp.float32) for _ in range(NBUF)],
        [pltpu.SemaphoreType.DMA for _ in range(NBUF)],
        [pltpu.SemaphoreType.DMA for _ in range(NBUF)],
        [pltpu.SemaphoreType.DMA for _ in range(NBUF)],
        pltpu.VMEM_SHARED((N_PAD, D_IN), jnp.float32),
    ],
    name="sage_agg",
)

_deg_call = pl.kernel(
    _deg_body,
    out_type=[
        jax.ShapeDtypeStruct((N_PAD,), jnp.float32),
        jax.ShapeDtypeStruct((N_PAD,), jnp.float32),
    ],
    mesh=_MESH,
    scratch_types=[
        pltpu.VMEM((CHUNK,), jnp.int32),
        pltpu.VMEM((CHUNK,), jnp.float32),
        pltpu.VMEM_SHARED((N_PAD,), jnp.float32),
    ],
    name="sage_deg",
)


_BM = 1264
_GRID = N_PAD // _BM


def _mlp_body(a0, a1, xr, d0, d1, w1, b1r, w2, g_o):
    recip = 1.0 / (d0[...] + d1[...] + 1.0)
    mean1 = (a0[...] + a1[...] + xr[...]) * recip
    h = jnp.dot(mean1, w1[...], preferred_element_type=jnp.float32) + b1r[...]
    h = jnp.maximum(h, 0.0)
    g_o[...] = jnp.dot(h, w2[...], preferred_element_type=jnp.float32)


def _mlp(agg0, agg1, x_p, d0, d1, W1, b1r, W2):
    return pl.pallas_call(
        _mlp_body,
        grid=(_GRID,),
        in_specs=[
            pl.BlockSpec((_BM, D_IN), lambda i: (i, 0)),
            pl.BlockSpec((_BM, D_IN), lambda i: (i, 0)),
            pl.BlockSpec((_BM, D_IN), lambda i: (i, 0)),
            pl.BlockSpec((_BM, 1), lambda i: (i, 0)),
            pl.BlockSpec((_BM, 1), lambda i: (i, 0)),
            pl.BlockSpec((D_IN, D_HID), lambda i: (0, 0)),
            pl.BlockSpec((1, D_HID), lambda i: (0, 0)),
            pl.BlockSpec((D_HID, D_OUT), lambda i: (0, 0)),
        ],
        out_specs=pl.BlockSpec((_BM, D_OUT), lambda i: (i, 0)),
        out_shape=jax.ShapeDtypeStruct((N_PAD, D_OUT), jnp.float32),
    )(agg0, agg1, x_p, d0, d1, W1, b1r, W2)


def _final_body(a0, a1, gr, d0, d1, b2r, o):
    recip = 1.0 / (d0[...] + d1[...] + 1.0)
    z = (a0[...] + a1[...] + gr[...]) * recip + b2r[...]
    m = jnp.max(z, axis=1, keepdims=True)
    zs = z - m
    lse = jnp.log(jnp.sum(jnp.exp(zs), axis=1, keepdims=True))
    o[...] = zs - lse


def _final(a0, a1, g, d0, d1, b2r):
    return pl.pallas_call(
        _final_body,
        grid=(_GRID,),
        in_specs=[
            pl.BlockSpec((_BM, D_OUT), lambda i: (i, 0)),
            pl.BlockSpec((_BM, D_OUT), lambda i: (i, 0)),
            pl.BlockSpec((_BM, D_OUT), lambda i: (i, 0)),
            pl.BlockSpec((_BM, 1), lambda i: (i, 0)),
            pl.BlockSpec((_BM, 1), lambda i: (i, 0)),
            pl.BlockSpec((1, D_OUT), lambda i: (0, 0)),
        ],
        out_specs=pl.BlockSpec((_BM, D_OUT), lambda i: (i, 0)),
        out_shape=jax.ShapeDtypeStruct((N_PAD, D_OUT), jnp.float32),
    )(a0, a1, g, d0, d1, b2r)


def kernel(x, edge_index, W1, b1, W2, b2):
    src = edge_index[0]
    dst = edge_index[1]
    pad = E_PAD - N_EDGES
    fill = jnp.arange(pad, dtype=jnp.int32)
    src_p = jnp.concatenate([src, fill % N_NODES])
    dst_p = jnp.concatenate([dst, N_NODES + fill % (N_PAD - N_NODES)])
    e3 = jnp.stack([src_p.reshape(-1, CHUNK), dst_p.reshape(-1, CHUNK)],
                   axis=1)
    zf = jnp.zeros((N_PAD, D_IN), jnp.float32)
    zd = jnp.zeros((N_PAD,), jnp.float32)

    agg0, agg1 = _agg_call(x, e3, zf)
    deg0, deg1 = _deg_call(dst_p, zd)
    d0 = deg0[:, None]
    d1 = deg1[:, None]
    x_p = jnp.pad(x, ((0, N_PAD - N_NODES), (0, 0)))

    g = _mlp(agg0, agg1, x_p, d0, d1, W1, b1.reshape(1, -1), W2)

    agg2_0, agg2_1 = _agg_call(g, e3, zf)

    out = _final(agg2_0, agg2_1, g, d0, d1, b2.reshape(1, -1))
    return out[:N_NODES]


# speedup vs baseline: 19.9219x; 1.2373x over previous
import jax
import jax.numpy as jnp
from jax import lax
from jax.experimental import pallas as pl
from jax.experimental.pallas import tpu as pltpu
from jax.experimental.pallas import tpu_sc as plsc

N_NODES = 10000
N_EDGES = 320000
D_IN = 128
D_HID = 256
D_OUT = 128

N_TILES = 16
N_SC = 2
NW = N_TILES * N_SC
CHUNK = 64
ROWS_PER_TILE = 632
N_PAD = N_TILES * ROWS_PER_TILE
NBUF = 5
GROUPS = 32
CHUNKS_PER_TILE = NBUF * GROUPS
E_PAD = NW * CHUNK * CHUNKS_PER_TILE

_MESH = plsc.VectorSubcoreMesh(core_axis_name="c", subcore_axis_name="s")

_SEGS = [(o, min(CHUNK, ROWS_PER_TILE - o)) for o in range(0, ROWS_PER_TILE, CHUNK)]


def _bounce(src_ref, dst_ref, bounce_ref, rbase):
    for s_off, s_len in _SEGS:
        sl = pl.ds(rbase + s_off, s_len)
        pltpu.sync_copy(src_ref.at[sl], bounce_ref.at[pl.ds(0, s_len)])
        pltpu.sync_copy(bounce_ref.at[pl.ds(0, s_len)], dst_ref.at[sl])


def _vcopy_row(src2d, row, dst1d):
    for j in range(CHUNK // 16):
        dst1d[pl.ds(j * 16, 16)] = src2d[row, pl.ds(j * 16, 16)]


def _agg_body(feat, e3, zf,
              agg0_o, agg1_o,
              idx, gidx, sidx, rows, si, sg, ss, agg_sh):
    cid = lax.axis_index("c")
    tid = lax.axis_index("s")
    wid = tid * N_SC + cid
    rbase = tid * ROWS_PER_TILE
    _bounce(zf, agg_sh, rows[0], rbase)
    plsc.subcore_barrier()

    cbase = wid * CHUNKS_PER_TILE

    def _wait_idx(b):
        pltpu.make_async_copy(e3.at[0], idx[b], si[b]).wait()

    def _wait_gather(b):
        pltpu.make_async_copy(zf.at[pl.ds(0, CHUNK)], rows[b], sg[b]).wait()

    def _wait_scatter(b):
        pltpu.make_async_copy(zf.at[pl.ds(0, CHUNK)], rows[b], ss[b]).wait()

    def _group(g, first, issue_next):
        for b in range(NBUF):
            k = cbase + g * NBUF + b
            if not first:
                _wait_scatter(b)
            _wait_idx(b)
            _vcopy_row(idx[b], 0, gidx[b])
            _vcopy_row(idx[b], 1, sidx[b])
            if issue_next:
                pltpu.async_copy(e3.at[k + NBUF], idx[b], si[b])
            pltpu.async_copy(feat.at[gidx[b]], rows[b], sg[b])
        for b in range(NBUF):
            _wait_gather(b)
            pltpu.async_copy(rows[b], agg_sh.at[sidx[b]], ss[b], add=True)

    for b in range(NBUF):
        pltpu.async_copy(e3.at[cbase + b], idx[b], si[b])
    _group(0, True, True)
    lax.fori_loop(1, GROUPS - 1,
                  lambda g, c: (_group(g, False, True), c)[1], None)
    _group(GROUPS - 1, False, False)
    for b in range(NBUF):
        _wait_scatter(b)
    plsc.subcore_barrier()

    @pl.when(cid == 0)
    def _():
        _bounce(agg_sh, agg0_o, rows[0], rbase)

    @pl.when(cid == 1)
    def _():
        _bounce(agg_sh, agg1_o, rows[0], rbase)


CHUNK_D = 128
DBUF = 4
CHUNKS_D = E_PAD // (NW * CHUNK_D)
GROUPS_D = CHUNKS_D // DBUF


def _deg_body(dst, zd,
              deg0_o, deg1_o,
              dstb, ones_v, si, ss, deg_sh):
    cid = lax.axis_index("c")
    tid = lax.axis_index("s")
    wid = tid * N_SC + cid
    rbase = tid * ROWS_PER_TILE
    _bounce(zd, deg_sh, ones_v, rbase)
    for j in range(CHUNK_D // 16):
        ones_v[pl.ds(j * 16, 16)] = jnp.ones((16,), jnp.float32)
    plsc.subcore_barrier()

    cbase = wid * CHUNKS_D

    def _idx_issue(p, b, k):
        pltpu.async_copy(dst.at[pl.ds((cbase + k) * CHUNK_D, CHUNK_D)],
                         dstb[p][b], si[p][b])

    def _wait_idx(p, b):
        pltpu.make_async_copy(dst.at[pl.ds(0, CHUNK_D)], dstb[p][b],
                              si[p][b]).wait()

    def _wait_scat(b):
        pltpu.make_async_copy(zd.at[pl.ds(0, CHUNK_D)], ones_v, ss[b]).wait()

    def _dgroup(g, p, first, issue_next):
        for b in range(DBUF):
            if not first:
                _wait_scat(b)
            if issue_next:
                _idx_issue(1 - p, b, (g + 1) * DBUF + b)
            _wait_idx(p, b)
            pltpu.async_copy(ones_v, deg_sh.at[dstb[p][b]], ss[b], add=True)

    for b in range(DBUF):
        _idx_issue(0, b, b)
    _dgroup(0, 0, True, True)
    _dgroup(1, 1, False, True)

    def _two(gp, carry):
        g = 2 * gp
        _dgroup(g, 0, False, True)
        _dgroup(g + 1, 1, False, True)
        return carry

    lax.fori_loop(1, GROUPS_D // 2 - 1, _two, None)
    _dgroup(GROUPS_D - 2, 0, False, True)
    _dgroup(GROUPS_D - 1, 1, False, False)
    for b in range(DBUF):
        _wait_scat(b)
    plsc.subcore_barrier()

    @pl.when(cid == 0)
    def _():
        _bounce(deg_sh, deg0_o, ones_v, rbase)

    @pl.when(cid == 1)
    def _():
        _bounce(deg_sh, deg1_o, ones_v, rbase)


_agg_call = pl.kernel(
    _agg_body,
    out_type=[
        jax.ShapeDtypeStruct((N_PAD, D_IN), jnp.float32),
        jax.ShapeDtypeStruct((N_PAD, D_IN), jnp.float32),
    ],
    mesh=_MESH,
    scratch_types=[
        [pltpu.VMEM((2, CHUNK), jnp.int32) for _ in range(NBUF)],
        [pltpu.VMEM((CHUNK,), jnp.int32) for _ in range(NBUF)],
        [pltpu.VMEM((CHUNK,), jnp.int32) for _ in range(NBUF)],
        [pltpu.VMEM((CHUNK, D_IN), jnp.float32) for _ in range(NBUF)],
        [pltpu.SemaphoreType.DMA for _ in range(NBUF)],
        [pltpu.SemaphoreType.DMA for _ in range(NBUF)],
        [pltpu.SemaphoreType.DMA for _ in range(NBUF)],
        pltpu.VMEM_SHARED((N_PAD, D_IN), jnp.float32),
    ],
    name="sage_agg",
)

_deg_call = pl.kernel(
    _deg_body,
    out_type=[
        jax.ShapeDtypeStruct((N_PAD,), jnp.float32),
        jax.ShapeDtypeStruct((N_PAD,), jnp.float32),
    ],
    mesh=_MESH,
    scratch_types=[
        [[pltpu.VMEM((CHUNK_D,), jnp.int32) for _ in range(DBUF)]
         for _ in range(2)],
        pltpu.VMEM((CHUNK_D,), jnp.float32),
        [[pltpu.SemaphoreType.DMA for _ in range(DBUF)] for _ in range(2)],
        [pltpu.SemaphoreType.DMA for _ in range(DBUF)],
        pltpu.VMEM_SHARED((N_PAD,), jnp.float32),
    ],
    name="sage_deg",
)


_BM = 1264
_GRID = N_PAD // _BM


def _mlp_body(a0, a1, xr, d0, d1, w1, b1r, w2, g_o):
    recip = 1.0 / (d0[...] + d1[...] + 1.0)
    mean1 = (a0[...] + a1[...] + xr[...]) * recip
    h = jnp.dot(mean1, w1[...], preferred_element_type=jnp.float32) + b1r[...]
    h = jnp.maximum(h, 0.0)
    g_o[...] = jnp.dot(h, w2[...], preferred_element_type=jnp.float32)


def _mlp(agg0, agg1, x_p, d0, d1, W1, b1r, W2):
    return pl.pallas_call(
        _mlp_body,
        grid=(_GRID,),
        in_specs=[
            pl.BlockSpec((_BM, D_IN), lambda i: (i, 0)),
            pl.BlockSpec((_BM, D_IN), lambda i: (i, 0)),
            pl.BlockSpec((_BM, D_IN), lambda i: (i, 0)),
            pl.BlockSpec((_BM, 1), lambda i: (i, 0)),
            pl.BlockSpec((_BM, 1), lambda i: (i, 0)),
            pl.BlockSpec((D_IN, D_HID), lambda i: (0, 0)),
            pl.BlockSpec((1, D_HID), lambda i: (0, 0)),
            pl.BlockSpec((D_HID, D_OUT), lambda i: (0, 0)),
        ],
        out_specs=pl.BlockSpec((_BM, D_OUT), lambda i: (i, 0)),
        out_shape=jax.ShapeDtypeStruct((N_PAD, D_OUT), jnp.float32),
    )(agg0, agg1, x_p, d0, d1, W1, b1r, W2)


def _final_body(a0, a1, gr, d0, d1, b2r, o):
    recip = 1.0 / (d0[...] + d1[...] + 1.0)
    z = (a0[...] + a1[...] + gr[...]) * recip + b2r[...]
    m = jnp.max(z, axis=1, keepdims=True)
    zs = z - m
    lse = jnp.log(jnp.sum(jnp.exp(zs), axis=1, keepdims=True))
    o[...] = zs - lse


def _final(a0, a1, g, d0, d1, b2r):
    return pl.pallas_call(
        _final_body,
        grid=(_GRID,),
        in_specs=[
            pl.BlockSpec((_BM, D_OUT), lambda i: (i, 0)),
            pl.BlockSpec((_BM, D_OUT), lambda i: (i, 0)),
            pl.BlockSpec((_BM, D_OUT), lambda i: (i, 0)),
            pl.BlockSpec((_BM, 1), lambda i: (i, 0)),
            pl.BlockSpec((_BM, 1), lambda i: (i, 0)),
            pl.BlockSpec((1, D_OUT), lambda i: (0, 0)),
        ],
        out_specs=pl.BlockSpec((_BM, D_OUT), lambda i: (i, 0)),
        out_shape=jax.ShapeDtypeStruct((N_PAD, D_OUT), jnp.float32),
    )(a0, a1, g, d0, d1, b2r)


def kernel(x, edge_index, W1, b1, W2, b2):
    src = edge_index[0]
    dst = edge_index[1]
    pad = E_PAD - N_EDGES
    fill = jnp.arange(pad, dtype=jnp.int32)
    src_p = jnp.concatenate([src, fill % N_NODES])
    dst_p = jnp.concatenate([dst, N_NODES + fill % (N_PAD - N_NODES)])
    e3 = jnp.stack([src_p.reshape(-1, CHUNK), dst_p.reshape(-1, CHUNK)],
                   axis=1)
    zf = jnp.zeros((N_PAD, D_IN), jnp.float32)
    zd = jnp.zeros((N_PAD,), jnp.float32)

    agg0, agg1 = _agg_call(x, e3, zf)
    deg0, deg1 = _deg_call(dst_p, zd)
    d0 = deg0[:, None]
    d1 = deg1[:, None]
    x_p = jnp.pad(x, ((0, N_PAD - N_NODES), (0, 0)))

    g = _mlp(agg0, agg1, x_p, d0, d1, W1, b1.reshape(1, -1), W2)

    agg2_0, agg2_1 = _agg_call(g, e3, zf)

    out = _final(agg2_0, agg2_1, g, d0, d1, b2.reshape(1, -1))
    return out[:N_NODES]
